# baseline (speedup 1.0000x reference)
import functools

import jax
import jax.numpy as jnp
from jax import lax
from jax.experimental import pallas as pl
from jax.experimental.pallas import tpu as pltpu
from jax.experimental.pallas import tpu_sc as plsc

_NS = 16
_CH = 128
_L = 16
_R = 2
_RG = 3
_RE = 8

_GDN = lax.GatherDimensionNumbers(
    offset_dims=(), collapsed_slice_dims=(0,), start_index_map=(0,))


def _bcast_lane(v16, i):
    idx = jnp.full((_L, 1), i, jnp.int32)
    return lax.gather(v16, idx, dimension_numbers=_GDN, slice_sizes=(1,),
                      mode=lax.GatherScatterMode.PROMISE_IN_BOUNDS)


@functools.partial(jax.jit, static_argnames=("n", "h", "nch", "epc", "ndup"))
def _sc_agg(row2, col2, w2, x, zeros, *, n, h, nch, epc, ndup):
    mesh = plsc.VectorSubcoreMesh(core_axis_name="c", subcore_axis_name="s")
    rps = n // _NS

    @functools.partial(
        pl.kernel,
        out_type=jax.ShapeDtypeStruct((n, 2 * h), jnp.float32),
        mesh=mesh,
        scratch_types=[
            pltpu.VMEM((_RE, _CH), jnp.int32),
            pltpu.VMEM((_RE, _CH), jnp.int32),
            pltpu.VMEM((_RE, _CH), jnp.float32),
            pltpu.VMEM((_R, _CH), jnp.int32),
            pltpu.VMEM((_RG, _CH, h), jnp.float32),
            pltpu.VMEM((_R, _CH, h), jnp.float32),
            pltpu.VMEM_SHARED((n, h), jnp.float32),
            pltpu.VMEM_SHARED((n, h), jnp.float32),
            pltpu.SemaphoreType.DMA((_RE,)),
            pltpu.SemaphoreType.DMA((_RG,)),
            pltpu.SemaphoreType.DMA((_R,)),
        ],
        compiler_params=pltpu.CompilerParams(use_tc_tiling_on_sc=False),
    )
    def body(row_h, col_h, w_h, x_h, zeros_h, out_h,
             ecol, erow, ew, rid, rows_g, rows_s, xsh, acc,
             esem, gsem, ssem):
        c = lax.axis_index("c")
        s = lax.axis_index("s")

        pltpu.sync_copy(x_h.at[pl.ds(s * rps, rps), pl.ds(c * h, h)],
                        xsh.at[pl.ds(s * rps, rps)])
        pltpu.sync_copy(zeros_h, acc.at[pl.ds(s * rps, rps)])
        plsc.subcore_barrier()

        def edge_dmas(j, d):
            off = jnp.minimum(j * _CH, epc - _CH)
            sl = pl.ds(off, _CH)
            return (
                pltpu.make_async_copy(col_h.at[s, sl], ecol.at[d], esem.at[d]),
                pltpu.make_async_copy(row_h.at[s, sl], erow.at[d], esem.at[d]),
                pltpu.make_async_copy(w_h.at[s, sl], ew.at[d], esem.at[d]),
            )

        def gather(b, d):
            return pltpu.make_async_copy(
                xsh.at[ecol.at[d]], rows_g.at[b], gsem.at[b])

        def scatter(b):
            return pltpu.make_async_copy(
                rows_s.at[b], acc.at[rid.at[b]], ssem.at[b])

        for k in range(_RE):
            for dma in edge_dmas(k, k):
                dma.start()
        for k in range(_RG - 1):
            for dma in edge_dmas(k, k):
                dma.wait()
            gather(k, k).start()

        def step(g, carry):
            bg = lax.rem(g, _RG)
            b = lax.rem(g, _R)
            d = lax.rem(g, _RE)

            dn = lax.rem(g + _RG - 1, _RE)
            for dma in edge_dmas(0, dn):
                dma.wait()
            gather(lax.rem(g + _RG - 1, _RG), dn).start()

            gather(bg, d).wait()

            @pl.when(g >= _R)
            def _():
                scatter(b).wait()

            for q in range(_CH // _L):
                sl = pl.ds(_L * q, _L)
                rid[b, sl] = erow[d, sl]
            last_gap = nch - 1 - g
            for grp in range(_CH // _L):
                w16 = ew[d, pl.ds(_L * grp, _L)]
                if ndup > 0 and _L * grp < ndup:
                    offs = (lax.iota(jnp.int32, _L) + (_L * grp - ndup)
                            + last_gap * _CH)
                    fac = jnp.clip(offs.astype(jnp.float32) + 1.0, 0.0, 1.0)
                    w16 = w16 * fac
                for i in range(_L):
                    e = _L * grp + i
                    wb = _bcast_lane(w16, i)
                    for q in range(h // _L):
                        sl = pl.ds(_L * q, _L)
                        rows_s[b, e, sl] = rows_g[bg, e, sl] * wb
            scatter(b).start(add=True)

            for dma in edge_dmas(jnp.minimum(g + _RE, nch - 1), d):
                dma.start()
            return carry

        lax.fori_loop(0, nch, step, 0)

        for m in range(_RG):
            fires = int(m < _RG - 1) + sum(
                1 for g in range(nch) if (g + _RG - 1) % _RG == m)
            waits = sum(1 for g in range(nch) if g % _RG == m)
            assert fires >= waits
            for _ in range(fires - waits):
                gather(m, 0).wait()
        for b in range(_R):
            scatter(b).wait()
        for k in range(_RE):
            fires = 1 + sum(1 for g in range(nch) if g % _RE == k)
            waits = int(k < _RG - 1) + sum(
                1 for g in range(nch) if (g + _RG - 1) % _RE == k)
            assert fires >= waits
            for _ in range(fires - waits):
                for dma in edge_dmas(0, k):
                    dma.wait()
        plsc.subcore_barrier()
        pltpu.sync_copy(acc.at[pl.ds(s * rps, rps)],
                        out_h.at[pl.ds(s * rps, rps), pl.ds(c * h, h)])

    return body(row2, col2, w2, x, zeros)


def kernel(edge_index, edge_weight, x):
    n, d = x.shape
    e = edge_weight.shape[0]
    h = d // 2

    row = edge_index[0].astype(jnp.int32)
    col = edge_index[1].astype(jnp.int32)
    w = edge_weight.astype(jnp.float32)

    unit = _NS * 8
    epad = ((e + unit - 1) // unit) * unit
    pad = epad - e
    if pad:
        row = jnp.concatenate([row, jnp.zeros((pad,), jnp.int32)])
        col = jnp.concatenate([col, jnp.zeros((pad,), jnp.int32)])
        w = jnp.concatenate([w, jnp.zeros((pad,), jnp.float32)])
    epc = epad // _NS
    nch = -(-epc // _CH)
    ndup = nch * _CH - epc

    row2 = row.reshape(_NS, epc)
    col2 = col.reshape(_NS, epc)
    w2 = w.reshape(_NS, epc)
    zeros = jnp.zeros((n // _NS, h), jnp.float32)

    return _sc_agg(row2, col2, w2, x, zeros,
                   n=n, h=h, nch=nch, epc=epc, ndup=ndup)


# speedup vs baseline: 3.3307x; 3.3307x over previous
import functools

import jax
import jax.numpy as jnp
from jax import lax
from jax.experimental import pallas as pl
from jax.experimental.pallas import tpu as pltpu
from jax.experimental.pallas import tpu_sc as plsc

_NS = 16
_CH = 128
_L = 16
_R = 2
_RE = 8

_GDN = lax.GatherDimensionNumbers(
    offset_dims=(), collapsed_slice_dims=(0,), start_index_map=(0,))


def _bcast_lane(v16, i):
    idx = jnp.full((_L, 1), i, jnp.int32)
    return lax.gather(v16, idx, dimension_numbers=_GDN, slice_sizes=(1,),
                      mode=lax.GatherScatterMode.PROMISE_IN_BOUNDS)


@functools.partial(jax.jit, static_argnames=("n", "h", "nch", "epc", "ndup"))
def _sc_agg(row2, col2, w2, x, zeros, *, n, h, nch, epc, ndup):
    mesh = plsc.VectorSubcoreMesh(core_axis_name="c", subcore_axis_name="s")
    rps = n // _NS

    @functools.partial(
        pl.kernel,
        out_type=jax.ShapeDtypeStruct((n, 2 * h), jnp.float32),
        mesh=mesh,
        scratch_types=[
            pltpu.VMEM((_RE, _CH), jnp.int32),
            pltpu.VMEM((_RE, _CH), jnp.int32),
            pltpu.VMEM((_RE, _CH), jnp.float32),
            pltpu.VMEM((_R, _CH), jnp.int32),
            pltpu.VMEM((_R, _CH, h), jnp.float32),
            pltpu.VMEM((_R, _CH, h), jnp.float32),
            pltpu.VMEM_SHARED((n, h), jnp.float32),
            pltpu.VMEM_SHARED((n, h), jnp.float32),
            pltpu.SemaphoreType.DMA((_RE,)),
            pltpu.SemaphoreType.DMA((_R,)),
            pltpu.SemaphoreType.DMA((_R,)),
        ],
        compiler_params=pltpu.CompilerParams(use_tc_tiling_on_sc=False),
    )
    def body(row_h, col_h, w_h, x_h, zeros_h, out_h,
             ecol, erow, ew, rid, rows_g, rows_s, xsh, acc,
             esem, gsem, ssem):
        c = lax.axis_index("c")
        s = lax.axis_index("s")

        pltpu.sync_copy(x_h.at[pl.ds(s * rps, rps), pl.ds(c * h, h)],
                        xsh.at[pl.ds(s * rps, rps)])
        pltpu.sync_copy(zeros_h, acc.at[pl.ds(s * rps, rps)])
        plsc.subcore_barrier()

        def edge_dmas(j, d):
            off = jnp.minimum(j * _CH, epc - _CH)
            sl = pl.ds(off, _CH)
            return (
                pltpu.make_async_copy(col_h.at[s, sl], ecol.at[d], esem.at[d]),
                pltpu.make_async_copy(row_h.at[s, sl], erow.at[d], esem.at[d]),
                pltpu.make_async_copy(w_h.at[s, sl], ew.at[d], esem.at[d]),
            )

        def gather(b, d):
            return pltpu.make_async_copy(
                xsh.at[ecol.at[d]], rows_g.at[b], gsem.at[b])

        def scatter(b):
            return pltpu.make_async_copy(
                rows_s.at[b], acc.at[rid.at[b]], ssem.at[b])

        for k in range(_RE):
            for dma in edge_dmas(k, k):
                dma.start()
        for k in range(_R):
            for dma in edge_dmas(k, k):
                dma.wait()
            gather(k, k).start()

        def step(g, carry):
            b = lax.rem(g, _R)
            d = lax.rem(g, _RE)
            gather(b, d).wait()

            @pl.when(g >= _R)
            def _():
                scatter(b).wait()

            for q in range(_CH // _L):
                sl = pl.ds(_L * q, _L)
                rid[b, sl] = erow[d, sl]
            last_gap = nch - 1 - g
            for grp in range(_CH // _L):
                w16 = ew[d, pl.ds(_L * grp, _L)]
                if ndup > 0 and _L * grp < ndup:
                    offs = (lax.iota(jnp.int32, _L) + (_L * grp - ndup)
                            + last_gap * _CH)
                    fac = jnp.clip(offs.astype(jnp.float32) + 1.0, 0.0, 1.0)
                    w16 = w16 * fac
                for i in range(_L):
                    e = _L * grp + i
                    wb = _bcast_lane(w16, i)
                    for q in range(h // _L):
                        sl = pl.ds(_L * q, _L)
                        rows_s[b, e, sl] = rows_g[b, e, sl] * wb
            scatter(b).start(add=True)

            for dma in edge_dmas(jnp.minimum(g + _RE, nch - 1), d):
                dma.start()

            dn = lax.rem(g + _R, _RE)
            for dma in edge_dmas(0, dn):
                dma.wait()
            gather(b, dn).start()
            return carry

        lax.fori_loop(0, nch, step, 0)

        for b in range(_R):
            gather(b, b).wait()
            scatter(b).wait()
        for k in range(_RE):
            fires = 1 + sum(1 for g in range(nch) if g % _RE == k)
            waits = int(k < _R) + sum(
                1 for g in range(nch) if (g + _R) % _RE == k)
            assert fires >= waits
            for _ in range(fires - waits):
                for dma in edge_dmas(0, k):
                    dma.wait()
        plsc.subcore_barrier()
        pltpu.sync_copy(acc.at[pl.ds(s * rps, rps)],
                        out_h.at[pl.ds(s * rps, rps), pl.ds(c * h, h)])

    return body(row2, col2, w2, x, zeros)


def kernel(edge_index, edge_weight, x):
    n, d = x.shape
    e = edge_weight.shape[0]
    h = d // 2

    row = edge_index[0].astype(jnp.int32)
    col = edge_index[1].astype(jnp.int32)
    w = edge_weight.astype(jnp.float32)

    unit = _NS * 8
    epad = ((e + unit - 1) // unit) * unit
    pad = epad - e
    if pad:
        row = jnp.concatenate([row, jnp.zeros((pad,), jnp.int32)])
        col = jnp.concatenate([col, jnp.zeros((pad,), jnp.int32)])
        w = jnp.concatenate([w, jnp.zeros((pad,), jnp.float32)])
    epc = epad // _NS
    nch = -(-epc // _CH)
    ndup = nch * _CH - epc

    row2 = row.reshape(_NS, epc)
    col2 = col.reshape(_NS, epc)
    w2 = w.reshape(_NS, epc)
    zeros = jnp.zeros((n // _NS, h), jnp.float32)

    return _sc_agg(row2, col2, w2, x, zeros,
                   n=n, h=h, nch=nch, epc=epc, ndup=ndup)


# speedup vs baseline: 3.5410x; 1.0632x over previous
import functools

import jax
import jax.numpy as jnp
from jax import lax
from jax.experimental import pallas as pl
from jax.experimental.pallas import tpu as pltpu
from jax.experimental.pallas import tpu_sc as plsc

_NS = 16
_CH = 128
_L = 16
_R = 2
_RE = 8

_GDN = lax.GatherDimensionNumbers(
    offset_dims=(), collapsed_slice_dims=(0,), start_index_map=(0,))


def _bcast_lane(v16, i):
    idx = jnp.full((_L, 1), i, jnp.int32)
    return lax.gather(v16, idx, dimension_numbers=_GDN, slice_sizes=(1,),
                      mode=lax.GatherScatterMode.PROMISE_IN_BOUNDS)


@functools.partial(jax.jit, static_argnames=("n", "h", "nch", "epc", "ndup"))
def _sc_agg(ei2, w1, x, zeros, *, n, h, nch, epc, ndup):
    mesh = plsc.VectorSubcoreMesh(core_axis_name="c", subcore_axis_name="s")
    rps = n // _NS

    @functools.partial(
        pl.kernel,
        out_type=jax.ShapeDtypeStruct((n, 2 * h), jnp.float32),
        mesh=mesh,
        scratch_types=[
            pltpu.VMEM((_RE, _CH), jnp.int32),
            pltpu.VMEM((_RE, _CH), jnp.int32),
            pltpu.VMEM((_RE, _CH), jnp.float32),
            pltpu.VMEM((_R, _CH), jnp.int32),
            pltpu.VMEM((_R, _CH, h), jnp.float32),
            pltpu.VMEM((_R, _CH, h), jnp.float32),
            pltpu.VMEM_SHARED((n, h), jnp.float32),
            pltpu.VMEM_SHARED((n, h), jnp.float32),
            pltpu.SemaphoreType.DMA((_RE,)),
            pltpu.SemaphoreType.DMA((_R,)),
            pltpu.SemaphoreType.DMA((_R,)),
        ],
        compiler_params=pltpu.CompilerParams(use_tc_tiling_on_sc=False),
    )
    def body(ei_h, w_h, x_h, zeros_h, out_h,
             ecol, erow, ew, rid, rows_g, rows_s, xsh, acc,
             esem, gsem, ssem):
        c = lax.axis_index("c")
        s = lax.axis_index("s")

        pltpu.sync_copy(x_h.at[pl.ds(s * rps, rps), pl.ds(c * h, h)],
                        xsh.at[pl.ds(s * rps, rps)])
        pltpu.sync_copy(zeros_h, acc.at[pl.ds(s * rps, rps)])
        plsc.subcore_barrier()

        def edge_dmas(j, d):
            base = s * epc + jnp.minimum(j * _CH, epc - _CH)
            sl = pl.ds(base, _CH)
            return (
                pltpu.make_async_copy(ei_h.at[1, sl], ecol.at[d], esem.at[d]),
                pltpu.make_async_copy(ei_h.at[0, sl], erow.at[d], esem.at[d]),
                pltpu.make_async_copy(w_h.at[sl], ew.at[d], esem.at[d]),
            )

        def gather(b, d):
            return pltpu.make_async_copy(
                xsh.at[ecol.at[d]], rows_g.at[b], gsem.at[b])

        def scatter(b):
            return pltpu.make_async_copy(
                rows_s.at[b], acc.at[rid.at[b]], ssem.at[b])

        for k in range(_RE):
            for dma in edge_dmas(k, k):
                dma.start()
        for k in range(_R):
            for dma in edge_dmas(k, k):
                dma.wait()
            gather(k, k).start()

        def step(g, carry):
            b = lax.rem(g, _R)
            d = lax.rem(g, _RE)
            gather(b, d).wait()

            @pl.when(g >= _R)
            def _():
                scatter(b).wait()

            for q in range(_CH // _L):
                sl = pl.ds(_L * q, _L)
                rid[b, sl] = erow[d, sl]
            last_gap = nch - 1 - g
            for grp in range(_CH // _L):
                w16 = ew[d, pl.ds(_L * grp, _L)]
                if ndup > 0 and _L * grp < ndup:
                    offs = (lax.iota(jnp.int32, _L) + (_L * grp - ndup)
                            + last_gap * _CH)
                    fac = jnp.clip(offs.astype(jnp.float32) + 1.0, 0.0, 1.0)
                    w16 = w16 * fac
                for i in range(_L):
                    e = _L * grp + i
                    wb = _bcast_lane(w16, i)
                    for q in range(h // _L):
                        sl = pl.ds(_L * q, _L)
                        rows_s[b, e, sl] = rows_g[b, e, sl] * wb
            scatter(b).start(add=True)

            for dma in edge_dmas(jnp.minimum(g + _RE, nch - 1), d):
                dma.start()

            dn = lax.rem(g + _R, _RE)
            for dma in edge_dmas(0, dn):
                dma.wait()
            gather(b, dn).start()
            return carry

        lax.fori_loop(0, nch, step, 0)

        for b in range(_R):
            gather(b, b).wait()
            scatter(b).wait()
        for k in range(_RE):
            fires = 1 + sum(1 for g in range(nch) if g % _RE == k)
            waits = int(k < _R) + sum(
                1 for g in range(nch) if (g + _R) % _RE == k)
            assert fires >= waits
            for _ in range(fires - waits):
                for dma in edge_dmas(0, k):
                    dma.wait()
        plsc.subcore_barrier()
        pltpu.sync_copy(acc.at[pl.ds(s * rps, rps)],
                        out_h.at[pl.ds(s * rps, rps), pl.ds(c * h, h)])

    return body(ei2, w1, x, zeros)


def kernel(edge_index, edge_weight, x):
    n, d = x.shape
    e = edge_weight.shape[0]
    h = d // 2

    ei = edge_index.astype(jnp.int32)
    w = edge_weight.astype(jnp.float32)

    unit = _NS * 8
    epad = ((e + unit - 1) // unit) * unit
    pad = epad - e
    if pad:
        ei = jnp.pad(ei, ((0, 0), (0, pad)))
        w = jnp.pad(w, (0, pad))
    epc = epad // _NS
    nch = -(-epc // _CH)
    ndup = nch * _CH - epc

    zeros = jnp.zeros((n // _NS, h), jnp.float32)

    return _sc_agg(ei, w, x, zeros, n=n, h=h, nch=nch, epc=epc, ndup=ndup)


# speedup vs baseline: 3.5916x; 1.0143x over previous
import functools

import jax
import jax.numpy as jnp
from jax import lax
from jax.experimental import pallas as pl
from jax.experimental.pallas import tpu as pltpu
from jax.experimental.pallas import tpu_sc as plsc

_NS = 16
_CH = 128
_L = 16
_R = 3
_RE = 4

_GDN = lax.GatherDimensionNumbers(
    offset_dims=(), collapsed_slice_dims=(0,), start_index_map=(0,))


def _bcast_lane(v16, i):
    idx = jnp.full((_L, 1), i, jnp.int32)
    return lax.gather(v16, idx, dimension_numbers=_GDN, slice_sizes=(1,),
                      mode=lax.GatherScatterMode.PROMISE_IN_BOUNDS)


@functools.partial(jax.jit, static_argnames=("n", "h", "nch", "epc", "ndup"))
def _sc_agg(ei2, w1, x, zeros, *, n, h, nch, epc, ndup):
    mesh = plsc.VectorSubcoreMesh(core_axis_name="c", subcore_axis_name="s")
    rps = n // _NS

    @functools.partial(
        pl.kernel,
        out_type=jax.ShapeDtypeStruct((n, 2 * h), jnp.float32),
        mesh=mesh,
        scratch_types=[
            pltpu.VMEM((_RE, _CH), jnp.int32),
            pltpu.VMEM((_RE, _CH), jnp.int32),
            pltpu.VMEM((_RE, _CH), jnp.float32),
            pltpu.VMEM((_R, _CH), jnp.int32),
            pltpu.VMEM((_R, _CH, h), jnp.float32),
            pltpu.VMEM((_R, _CH, h), jnp.float32),
            pltpu.VMEM_SHARED((n, h), jnp.float32),
            pltpu.VMEM_SHARED((n, h), jnp.float32),
            pltpu.SemaphoreType.DMA((_RE,)),
            pltpu.SemaphoreType.DMA((_R,)),
            pltpu.SemaphoreType.DMA((_R,)),
        ],
        compiler_params=pltpu.CompilerParams(use_tc_tiling_on_sc=False),
    )
    def body(ei_h, w_h, x_h, zeros_h, out_h,
             ecol, erow, ew, rid, rows_g, rows_s, xsh, acc,
             esem, gsem, ssem):
        c = lax.axis_index("c")
        s = lax.axis_index("s")

        pltpu.sync_copy(x_h.at[pl.ds(s * rps, rps), pl.ds(c * h, h)],
                        xsh.at[pl.ds(s * rps, rps)])
        pltpu.sync_copy(zeros_h, acc.at[pl.ds(s * rps, rps)])
        plsc.subcore_barrier()

        def edge_dmas(j, d):
            base = s * epc + jnp.minimum(j * _CH, epc - _CH)
            sl = pl.ds(base, _CH)
            return (
                pltpu.make_async_copy(ei_h.at[1, sl], ecol.at[d], esem.at[d]),
                pltpu.make_async_copy(ei_h.at[0, sl], erow.at[d], esem.at[d]),
                pltpu.make_async_copy(w_h.at[sl], ew.at[d], esem.at[d]),
            )

        def gather(b, d):
            return pltpu.make_async_copy(
                xsh.at[ecol.at[d]], rows_g.at[b], gsem.at[b])

        def scatter(b):
            return pltpu.make_async_copy(
                rows_s.at[b], acc.at[rid.at[b]], ssem.at[b])

        for k in range(_RE):
            for dma in edge_dmas(k, k):
                dma.start()
        for k in range(_R):
            for dma in edge_dmas(k, k):
                dma.wait()
            gather(k, k).start()

        def step(g, carry):
            b = lax.rem(g, _R)
            d = lax.rem(g, _RE)
            gather(b, d).wait()

            @pl.when(g >= _R)
            def _():
                scatter(b).wait()

            for q in range(_CH // _L):
                sl = pl.ds(_L * q, _L)
                rid[b, sl] = erow[d, sl]
            last_gap = nch - 1 - g
            for grp in range(_CH // _L):
                w16 = ew[d, pl.ds(_L * grp, _L)]
                if ndup > 0 and _L * grp < ndup:
                    offs = (lax.iota(jnp.int32, _L) + (_L * grp - ndup)
                            + last_gap * _CH)
                    fac = jnp.clip(offs.astype(jnp.float32) + 1.0, 0.0, 1.0)
                    w16 = w16 * fac
                for i in range(_L):
                    e = _L * grp + i
                    wb = _bcast_lane(w16, i)
                    for q in range(h // _L):
                        sl = pl.ds(_L * q, _L)
                        rows_s[b, e, sl] = rows_g[b, e, sl] * wb
            scatter(b).start(add=True)

            for dma in edge_dmas(jnp.minimum(g + _RE, nch - 1), d):
                dma.start()

            dn = lax.rem(g + _R, _RE)
            for dma in edge_dmas(0, dn):
                dma.wait()
            gather(b, dn).start()
            return carry

        lax.fori_loop(0, nch, step, 0)

        for b in range(_R):
            gather(b, b).wait()
            scatter(b).wait()
        for k in range(_RE):
            fires = 1 + sum(1 for g in range(nch) if g % _RE == k)
            waits = int(k < _R) + sum(
                1 for g in range(nch) if (g + _R) % _RE == k)
            assert fires >= waits
            for _ in range(fires - waits):
                for dma in edge_dmas(0, k):
                    dma.wait()
        plsc.subcore_barrier()
        pltpu.sync_copy(acc.at[pl.ds(s * rps, rps)],
                        out_h.at[pl.ds(s * rps, rps), pl.ds(c * h, h)])

    return body(ei2, w1, x, zeros)


def kernel(edge_index, edge_weight, x):
    n, d = x.shape
    e = edge_weight.shape[0]
    h = d // 2

    ei = edge_index.astype(jnp.int32)
    w = edge_weight.astype(jnp.float32)

    unit = _NS * 8
    epad = ((e + unit - 1) // unit) * unit
    pad = epad - e
    if pad:
        ei = jnp.pad(ei, ((0, 0), (0, pad)))
        w = jnp.pad(w, (0, pad))
    epc = epad // _NS
    nch = -(-epc // _CH)
    ndup = nch * _CH - epc

    zeros = jnp.zeros((n // _NS, h), jnp.float32)

    return _sc_agg(ei, w, x, zeros, n=n, h=h, nch=nch, epc=epc, ndup=ndup)
